# baseline (speedup 1.0000x reference)
import jax
import jax.numpy as jnp
from jax import lax
from jax.experimental import pallas as pl
from jax.experimental.pallas import tpu as pltpu
from jax.experimental.pallas import tpu_sc as plsc

B = 16384
F = 26
V = 100000
D = 16
NUM = 96
H = 512
EPS = 1e-5

NC = 2
NS = 16
NW = NC * NS
NPLANES = F * D
PLANES_PER_W = NPLANES // NW


OCH = 4096


def _gather_body(emb_t, cat_t, out_hbm, plane_v, idx_v, out_a, out_b,
                 sem_p, sem_oa, sem_ob):
    wid = lax.axis_index("s") * NC + lax.axis_index("c")
    out_bufs = (out_a, out_b)
    out_sems = (sem_oa, sem_ob)
    out_cps = [None, None]
    plane_cp = None

    for k in range(PLANES_PER_W):
        p = wid * PLANES_PER_W + k
        f = p // D
        d = p % D
        if k == 0:
            pltpu.sync_copy(emb_t.at[f, d], plane_v)
            pltpu.sync_copy(cat_t.at[f], idx_v)
        else:
            plane_cp.wait()

            @pl.when(f != (p - 1) // D)
            def _load_idx():
                pltpu.sync_copy(cat_t.at[f], idx_v)

        for c in range(B // OCH):
            buf = c % 2
            if out_cps[buf] is not None:
                out_cps[buf].wait()
                out_cps[buf] = None
            cur_out = out_bufs[buf]

            @plsc.parallel_loop(0, OCH, step=16, unroll=16)
            def _gather16(i, _c=c, _out=cur_out):
                _out[pl.ds(i, 16)] = plsc.load_gather(
                    plane_v, [idx_v[pl.ds(_c * OCH + i, 16)]])

            out_cps[buf] = pltpu.async_copy(
                cur_out, out_hbm.at[p, pl.ds(c * OCH, OCH)], out_sems[buf])
        if k < PLANES_PER_W - 1:
            p1 = p + 1
            plane_cp = pltpu.async_copy(emb_t.at[p1 // D, p1 % D], plane_v,
                                        sem_p)
    for cp in out_cps:
        if cp is not None:
            cp.wait()


_gather = pl.kernel(
    _gather_body,
    out_type=jax.ShapeDtypeStruct((NPLANES, B), jnp.float32),
    mesh=plsc.VectorSubcoreMesh(core_axis_name="c", subcore_axis_name="s"),
    scratch_types=[
        pltpu.VMEM((V,), jnp.float32),
        pltpu.VMEM((B,), jnp.int32),
        pltpu.VMEM((OCH,), jnp.float32),
        pltpu.VMEM((OCH,), jnp.float32),
        pltpu.SemaphoreType.DMA,
        pltpu.SemaphoreType.DMA,
        pltpu.SemaphoreType.DMA,
    ],
    compiler_params=pltpu.CompilerParams(needs_layout_passes=False),
)


NBC = 4096


def _mlp_body(xg, xn, w1a, w1b, b1r, w2f, b2r, w3f, b3r, out):
    xg_b = xg[...].astype(jnp.bfloat16)
    xn_b = xn[...].astype(jnp.bfloat16)
    z1 = jnp.dot(w1a[...], xg_b, preferred_element_type=jnp.float32)
    z1 = z1 + jnp.dot(w1b[...], xn_b, preferred_element_type=jnp.float32)
    z1 = jnp.maximum(z1 + b1r[...], 0.0).astype(jnp.bfloat16)
    z2 = jnp.dot(w2f[...], z1, preferred_element_type=jnp.float32)
    z2 = jnp.maximum(z2 + b2r[...], 0.0).astype(jnp.bfloat16)
    out[...] = jnp.dot(w3f[...], z2, preferred_element_type=jnp.float32) + b3r[...]


_mlp = pl.pallas_call(
    _mlp_body,
    grid=(B // NBC,),
    in_specs=[
        pl.BlockSpec((NPLANES, NBC), lambda i: (0, i)),
        pl.BlockSpec((NUM, NBC), lambda i: (0, i)),
        pl.BlockSpec((H, NPLANES), lambda i: (0, 0)),
        pl.BlockSpec((H, NUM), lambda i: (0, 0)),
        pl.BlockSpec((H, 1), lambda i: (0, 0)),
        pl.BlockSpec((H, H), lambda i: (0, 0)),
        pl.BlockSpec((H, 1), lambda i: (0, 0)),
        pl.BlockSpec((1, H), lambda i: (0, 0)),
        pl.BlockSpec((1, 1), lambda i: (0, 0)),
    ],
    out_specs=pl.BlockSpec((1, NBC), lambda i: (0, i)),
    out_shape=jax.ShapeDtypeStruct((1, B), jnp.float32),
)


def kernel(cat_data, num_data, emb, W1, b1, g1, bt1, W2, b2, g2, bt2, W3, b3):
    emb_t = jnp.transpose(emb, (0, 2, 1))
    cat_t = cat_data.T
    xn_t = num_data.T

    xg_t = _gather(emb_t, cat_t)

    s1 = g1 * (1.0 / jnp.sqrt(1.0 + EPS))
    s2 = g2 * (1.0 / jnp.sqrt(1.0 + EPS))
    w1a = W1[:, : F * D]
    w1b = W1[:, F * D :]
    w2f = W2 * s1[None, :]
    b2f = W2 @ bt1 + b2
    w3f = W3 * s2[None, :]
    b3f = W3 @ bt2 + b3

    out_row = _mlp(
        xg_t,
        xn_t,
        w1a.astype(jnp.bfloat16),
        w1b.astype(jnp.bfloat16),
        b1.reshape(H, 1),
        w2f.astype(jnp.bfloat16),
        b2f.reshape(H, 1),
        w3f.astype(jnp.bfloat16),
        b3f.reshape(1, 1),
    )
    return out_row.reshape(B, 1)


# speedup vs baseline: 1.0238x; 1.0238x over previous
import jax
import jax.numpy as jnp
from jax import lax
from jax.experimental import pallas as pl
from jax.experimental.pallas import tpu as pltpu
from jax.experimental.pallas import tpu_sc as plsc

B = 16384
F = 26
V = 100000
D = 16
NUM = 96
H = 512
EPS = 1e-5

NC = 2
NS = 16
NW = NC * NS
NPLANES = F * D
PLANES_PER_W = NPLANES // NW


OCH = 4096


def _gather_body(emb_t, cat_t, out_hbm, plane_v, idx_v, out_a, out_b,
                 sem_p, sem_oa, sem_ob):
    wid = lax.axis_index("s") * NC + lax.axis_index("c")
    out_bufs = (out_a, out_b)
    out_sems = (sem_oa, sem_ob)
    out_cps = [None, None]
    plane_cp = None

    for k in range(PLANES_PER_W):
        p = wid * PLANES_PER_W + k
        f = p // D
        d = p % D
        if k == 0:
            pltpu.sync_copy(emb_t.at[f, d], plane_v)
            pltpu.sync_copy(cat_t.at[f], idx_v)
        else:
            plane_cp.wait()

            @pl.when(f != (p - 1) // D)
            def _load_idx():
                pltpu.sync_copy(cat_t.at[f], idx_v)

        for c in range(B // OCH):
            buf = c % 2
            if out_cps[buf] is not None:
                out_cps[buf].wait()
                out_cps[buf] = None
            cur_out = out_bufs[buf]

            @plsc.parallel_loop(0, OCH, step=16, unroll=8)
            def _gather16(i, _c=c, _out=cur_out):
                _out[pl.ds(i, 16)] = plsc.load_gather(
                    plane_v, [idx_v[pl.ds(_c * OCH + i, 16)]])

            out_cps[buf] = pltpu.async_copy(
                cur_out, out_hbm.at[p, pl.ds(c * OCH, OCH)], out_sems[buf])
        if k < PLANES_PER_W - 1:
            p1 = p + 1
            plane_cp = pltpu.async_copy(emb_t.at[p1 // D, p1 % D], plane_v,
                                        sem_p)
    for cp in out_cps:
        if cp is not None:
            cp.wait()


_gather = pl.kernel(
    _gather_body,
    out_type=jax.ShapeDtypeStruct((NPLANES, B), jnp.float32),
    mesh=plsc.VectorSubcoreMesh(core_axis_name="c", subcore_axis_name="s"),
    scratch_types=[
        pltpu.VMEM((V,), jnp.float32),
        pltpu.VMEM((B,), jnp.int32),
        pltpu.VMEM((OCH,), jnp.float32),
        pltpu.VMEM((OCH,), jnp.float32),
        pltpu.SemaphoreType.DMA,
        pltpu.SemaphoreType.DMA,
        pltpu.SemaphoreType.DMA,
    ],
    compiler_params=pltpu.CompilerParams(needs_layout_passes=False),
)


NBC = 4096


def _mlp_body(xg, xn, w1a, w1b, b1r, w2f, b2r, w3f, b3r, out):
    xg_b = xg[...].astype(jnp.bfloat16)
    xn_b = xn[...].astype(jnp.bfloat16)
    z1 = jnp.dot(w1a[...], xg_b, preferred_element_type=jnp.float32)
    z1 = z1 + jnp.dot(w1b[...], xn_b, preferred_element_type=jnp.float32)
    z1 = jnp.maximum(z1 + b1r[...], 0.0).astype(jnp.bfloat16)
    z2 = jnp.dot(w2f[...], z1, preferred_element_type=jnp.float32)
    z2 = jnp.maximum(z2 + b2r[...], 0.0).astype(jnp.bfloat16)
    out[...] = jnp.dot(w3f[...], z2, preferred_element_type=jnp.float32) + b3r[...]


_mlp = pl.pallas_call(
    _mlp_body,
    grid=(B // NBC,),
    in_specs=[
        pl.BlockSpec((NPLANES, NBC), lambda i: (0, i)),
        pl.BlockSpec((NUM, NBC), lambda i: (0, i)),
        pl.BlockSpec((H, NPLANES), lambda i: (0, 0)),
        pl.BlockSpec((H, NUM), lambda i: (0, 0)),
        pl.BlockSpec((H, 1), lambda i: (0, 0)),
        pl.BlockSpec((H, H), lambda i: (0, 0)),
        pl.BlockSpec((H, 1), lambda i: (0, 0)),
        pl.BlockSpec((1, H), lambda i: (0, 0)),
        pl.BlockSpec((1, 1), lambda i: (0, 0)),
    ],
    out_specs=pl.BlockSpec((1, NBC), lambda i: (0, i)),
    out_shape=jax.ShapeDtypeStruct((1, B), jnp.float32),
)


def kernel(cat_data, num_data, emb, W1, b1, g1, bt1, W2, b2, g2, bt2, W3, b3):
    emb_t = jnp.transpose(emb, (0, 2, 1))
    cat_t = cat_data.T
    xn_t = num_data.T

    xg_t = _gather(emb_t, cat_t)

    s1 = g1 * (1.0 / jnp.sqrt(1.0 + EPS))
    s2 = g2 * (1.0 / jnp.sqrt(1.0 + EPS))
    w1a = W1[:, : F * D]
    w1b = W1[:, F * D :]
    w2f = W2 * s1[None, :]
    b2f = W2 @ bt1 + b2
    w3f = W3 * s2[None, :]
    b3f = W3 @ bt2 + b3

    out_row = _mlp(
        xg_t,
        xn_t,
        w1a.astype(jnp.bfloat16),
        w1b.astype(jnp.bfloat16),
        b1.reshape(H, 1),
        w2f.astype(jnp.bfloat16),
        b2f.reshape(H, 1),
        w3f.astype(jnp.bfloat16),
        b3f.reshape(1, 1),
    )
    return out_row.reshape(B, 1)
